# Initial kernel scaffold; baseline (speedup 1.0000x reference)
import jax
import jax.numpy as jnp
from jax.experimental import pallas as pl


def kernel(time, embeddings):
    raise NotImplementedError("write your pallas kernel here")


# speedup vs baseline: 1.7267x; 1.7267x over previous
import functools

import jax
import jax.numpy as jnp
from jax import lax
from jax.experimental import pallas as pl
from jax.experimental.pallas import tpu as pltpu
from jax.experimental.pallas import tpu_sc as plsc

_TABLE_ROWS = 2048
_DIM = 2048
_BATCH = 16384

_info = plsc.get_sparse_core_info()
_NC = _info.num_cores
_NS = _info.num_subcores
_NW = _NC * _NS
_BPW = _BATCH // _NW
_CHUNK = 32
_NCHUNK = _BPW // _CHUNK

_mesh = plsc.VectorSubcoreMesh(core_axis_name="c", subcore_axis_name="s")


@functools.partial(
    pl.kernel,
    mesh=_mesh,
    out_type=jax.ShapeDtypeStruct((_BATCH, _DIM), jnp.float32),
    scratch_types=[
        pltpu.VMEM((_BPW,), jnp.int32),
        pltpu.VMEM((_CHUNK, _DIM), jnp.float32),
        pltpu.SemaphoreType.DMA,
    ],
)
def _sc_gather(table_hbm, idx_hbm, out_hbm, idx_v, rows_v, gsem):
    wid = lax.axis_index("s") * _NC + lax.axis_index("c")
    base = wid * _BPW
    pltpu.sync_copy(idx_hbm.at[pl.ds(base, _BPW)], idx_v)

    def body(j, carry):
        off = j * _CHUNK
        pltpu.async_copy(
            table_hbm.at[idx_v.at[pl.ds(off, _CHUNK)]], rows_v, gsem
        ).wait()
        pltpu.sync_copy(rows_v, out_hbm.at[pl.ds(base + off, _CHUNK)])
        return carry

    lax.fori_loop(0, _NCHUNK, body, 0)


def kernel(time, embeddings):
    idx = time.astype(jnp.int32)
    return _sc_gather(embeddings, idx)


# speedup vs baseline: 1.7833x; 1.0328x over previous
import functools

import jax
import jax.numpy as jnp
from jax import lax
from jax.experimental import pallas as pl
from jax.experimental.pallas import tpu as pltpu
from jax.experimental.pallas import tpu_sc as plsc

_TABLE_ROWS = 2048
_DIM = 2048
_BATCH = 16384

_info = plsc.get_sparse_core_info()
_NC = _info.num_cores
_NS = _info.num_subcores
_NW = _NC * _NS
_BPW = _BATCH // _NW
_CHUNK = 16
_NCHUNK = _BPW // _CHUNK

_mesh = plsc.VectorSubcoreMesh(core_axis_name="c", subcore_axis_name="s")


@functools.partial(
    pl.kernel,
    mesh=_mesh,
    out_type=jax.ShapeDtypeStruct((_BATCH, _DIM), jnp.float32),
    scratch_types=[
        pltpu.VMEM((_BPW,), jnp.int32),
        pltpu.VMEM((_CHUNK, _DIM), jnp.float32),
        pltpu.VMEM((_CHUNK, _DIM), jnp.float32),
        pltpu.SemaphoreType.DMA,
        pltpu.SemaphoreType.DMA,
        pltpu.SemaphoreType.DMA,
        pltpu.SemaphoreType.DMA,
    ],
)
def _sc_gather(table_hbm, idx_hbm, out_hbm, idx_v, buf0, buf1, gs0, gs1,
               os0, os1):
    wid = lax.axis_index("s") * _NC + lax.axis_index("c")
    base = wid * _BPW
    pltpu.sync_copy(idx_hbm.at[pl.ds(base, _BPW)], idx_v)

    def gather(j, buf, sem):
        return pltpu.async_copy(
            table_hbm.at[idx_v.at[pl.ds(j * _CHUNK, _CHUNK)]], buf, sem)

    def scatter(j, buf, sem):
        return pltpu.async_copy(
            buf, out_hbm.at[pl.ds(base + j * _CHUNK, _CHUNK)], sem)

    gather(0, buf0, gs0)

    def body(i, carry):
        j = 2 * i
        gather(j + 1, buf1, gs1)
        pltpu.make_async_copy(
            table_hbm.at[idx_v.at[pl.ds(j * _CHUNK, _CHUNK)]], buf0, gs0
        ).wait()
        scatter(j, buf0, os0)
        pltpu.make_async_copy(
            table_hbm.at[idx_v.at[pl.ds((j + 1) * _CHUNK, _CHUNK)]], buf1, gs1
        ).wait()
        scatter(j + 1, buf1, os1)
        pltpu.make_async_copy(
            buf0, out_hbm.at[pl.ds(base + j * _CHUNK, _CHUNK)], os0).wait()

        @pl.when(j + 2 < _NCHUNK)
        def _():
            gather(j + 2, buf0, gs0)

        pltpu.make_async_copy(
            buf1, out_hbm.at[pl.ds(base + (j + 1) * _CHUNK, _CHUNK)], os1
        ).wait()
        return carry

    lax.fori_loop(0, _NCHUNK // 2, body, 0)


def kernel(time, embeddings):
    idx = time.astype(jnp.int32)
    return _sc_gather(embeddings, idx)
